# baseline (speedup 1.0000x reference)
import jax
import jax.numpy as jnp
from jax.experimental import pallas as pl
from jax.experimental.pallas import tpu as pltpu

_B = 1024
_K = 8
_N = 8192
_BB = 256
_BN = 2048


def _mm_kernel(c_ref, w_ref, b_ref, o_ref):
    c = c_ref[...]
    w = w_ref[...]
    acc = jax.lax.dot_general(c, w, (((1,), (1,)), ((), ())),
                              preferred_element_type=jnp.float32)
    o_ref[...] = acc + b_ref[...]


def kernel(condition, W, b):
    out = pl.pallas_call(
        _mm_kernel,
        grid=(_B // _BB, _N // _BN),
        in_specs=[
            pl.BlockSpec((_BB, _K), lambda i, j: (i, 0)),
            pl.BlockSpec((_BN, _K), lambda i, j: (j, 0)),
            pl.BlockSpec((1, _BN), lambda i, j: (0, j)),
        ],
        out_specs=pl.BlockSpec((_BB, _BN), lambda i, j: (i, j)),
        out_shape=jax.ShapeDtypeStruct((_B, _N), jnp.float32),
    )(condition, W, b.reshape(1, _N))
    return out.reshape(_B, 32, 16, 16)
